# baseline (speedup 1.0000x reference)
import functools

import jax
import jax.numpy as jnp
from jax import lax
from jax.experimental import pallas as pl
from jax.experimental.pallas import tpu as pltpu
from jax.experimental.pallas import tpu_sc as plsc

N = 10000
E = 320000
D = 128
H = 256

NC = 2
NS = 16
NW = NC * NS
CHUNK = 128
G = 8
S_STAGES = 10
CHUNKS_PER_W = G * S_STAGES
E_PER_W = CHUNK * CHUNKS_PER_W
E_PAD = E_PER_W * NW
ROWS_PER_S = 632
N_PAD = ROWS_PER_S * NS
TRASH = N + 8
EPS = 1e-5


def _sc_partial_segment_sum(x, idx_w, zeros):
    mesh = plsc.VectorSubcoreMesh(core_axis_name="c", subcore_axis_name="s")

    @functools.partial(
        pl.kernel,
        out_type=jax.ShapeDtypeStruct((NC * N_PAD, D), jnp.float32),
        mesh=mesh,
        scratch_types=[
            pltpu.VMEM((2, 2, G, CHUNK), jnp.int32),
            pltpu.VMEM((2, CHUNK, D), jnp.float32),
            pltpu.VMEM_SHARED((N_PAD, D), jnp.float32),
            pltpu.SemaphoreType.DMA,
            pltpu.SemaphoreType.DMA,
            pltpu.SemaphoreType.DMA,
            pltpu.SemaphoreType.DMA,
        ],
    )
    def sc_kernel(x_hbm, idx_hbm, z_hbm, out_hbm,
                  idx_i, rows_v, acc, is0, is1, rs0, rs1):
        c = lax.axis_index("c")
        s = lax.axis_index("s")
        wid = s * NC + c
        r0 = s * ROWS_PER_S
        isems = (is0, is1)
        rsems = (rs0, rs1)
        pltpu.sync_copy(z_hbm.at[pl.ds(r0, ROWS_PER_S)],
                        acc.at[pl.ds(r0, ROWS_PER_S)])
        pltpu.async_copy(idx_hbm.at[wid, 0], idx_i.at[0], isems[0])
        plsc.subcore_barrier()
        pltpu.make_async_copy(idx_hbm.at[wid, 0], idx_i.at[0], isems[0]).wait()
        for b in range(2):
            pltpu.async_copy(x_hbm.at[idx_i.at[0, 0, b]], rows_v.at[b], rsems[b])

        def pair_body(i, carry):
            for p in range(2):
                g = 2 * i + p
                nslot = 1 - p

                @pl.when(g + 1 < S_STAGES)
                def _():
                    pltpu.async_copy(idx_hbm.at[wid, g + 1],
                                     idx_i.at[nslot], isems[nslot])

                for b in range(G):
                    rb = b % 2
                    if b == G - 2:
                        @pl.when(g + 1 < S_STAGES)
                        def _():
                            pltpu.make_async_copy(
                                idx_hbm.at[wid, g + 1], idx_i.at[nslot],
                                isems[nslot]).wait()
                    pltpu.make_async_copy(x_hbm.at[idx_i.at[p, 0, b]],
                                          rows_v.at[rb], rsems[rb]).wait()
                    pltpu.sync_copy(rows_v.at[rb], acc.at[idx_i.at[p, 1, b]],
                                    add=True)
                    if b + 2 < G:
                        pltpu.async_copy(x_hbm.at[idx_i.at[p, 0, b + 2]],
                                         rows_v.at[rb], rsems[rb])
                    else:
                        @pl.when(g + 1 < S_STAGES)
                        def _():
                            pltpu.async_copy(
                                x_hbm.at[idx_i.at[nslot, 0, b + 2 - G]],
                                rows_v.at[rb], rsems[rb])
            return carry

        lax.fori_loop(0, S_STAGES // 2, pair_body, 0)
        plsc.subcore_barrier()
        out_base = c * N_PAD + r0
        pltpu.sync_copy(acc.at[pl.ds(r0, ROWS_PER_S)],
                        out_hbm.at[pl.ds(out_base, ROWS_PER_S)])

    return sc_kernel(x, idx_w, zeros)


def _tc_dense(x, p0, p1, W_root, W_nbr, b_base, gamma1, beta1,
              W1, b1, W2, b2, gamma2, beta2):
    def body(x_ref, p0_ref, p1_ref, wr_ref, wn_ref, bb_ref, g1_ref, be1_ref,
             w1_ref, b1_ref, w2_ref, b2_ref, g2_ref, be2_ref, o_ref):
        xv = x_ref[...]
        agg = p0_ref[...] + p1_ref[...]
        h = jnp.dot(xv, wr_ref[...], preferred_element_type=jnp.float32)
        h = h + jnp.dot(agg, wn_ref[...], preferred_element_type=jnp.float32)
        h = h + bb_ref[...]
        mu = jnp.mean(h, axis=0, keepdims=True)
        hc = h - mu
        var = jnp.mean(hc * hc, axis=0, keepdims=True)
        h = hc * lax.rsqrt(var + EPS) * g1_ref[...] + be1_ref[...]
        h = jnp.maximum(h, 0.0) + xv
        t = jnp.maximum(
            jnp.dot(h, w1_ref[...], preferred_element_type=jnp.float32)
            + b1_ref[...], 0.0)
        y = (jnp.dot(t, w2_ref[...], preferred_element_type=jnp.float32)
             + b2_ref[...] + h)
        mu2 = jnp.mean(y, axis=0, keepdims=True)
        yc = y - mu2
        var2 = jnp.mean(yc * yc, axis=0, keepdims=True)
        o_ref[...] = yc * lax.rsqrt(var2 + EPS) * g2_ref[...] + be2_ref[...]

    return pl.pallas_call(
        body,
        out_shape=jax.ShapeDtypeStruct((N, D), jnp.float32),
    )(x, p0, p1, W_root, W_nbr,
      b_base.reshape(1, D), gamma1.reshape(1, D), beta1.reshape(1, D),
      W1, b1.reshape(1, H), W2, b2.reshape(1, D),
      gamma2.reshape(1, D), beta2.reshape(1, D))


def kernel(x, edge_index, W_root, W_nbr, b_base, gamma1, beta1,
           W1, b1, W2, b2, gamma2, beta2):
    src = edge_index[0]
    dst = edge_index[1]
    pad = E_PAD - E
    src_w = jnp.concatenate(
        [src, jnp.zeros((pad,), jnp.int32)]).reshape(NW, S_STAGES, G, CHUNK)
    dst_w = jnp.concatenate(
        [dst, jnp.full((pad,), TRASH, jnp.int32)]).reshape(NW, S_STAGES, G, CHUNK)
    idx_w = jnp.stack([src_w, dst_w], axis=2)
    zeros = jnp.zeros((N_PAD, D), jnp.float32)
    parts = _sc_partial_segment_sum(x, idx_w, zeros)
    p0 = parts[:N]
    p1 = parts[N_PAD:N_PAD + N]
    return _tc_dense(x, p0, p1, W_root, W_nbr, b_base, gamma1, beta1,
                     W1, b1, W2, b2, gamma2, beta2)


# speedup vs baseline: 1.7197x; 1.7197x over previous
import functools

import jax
import jax.numpy as jnp
from jax import lax
from jax.experimental import pallas as pl
from jax.experimental.pallas import tpu as pltpu
from jax.experimental.pallas import tpu_sc as plsc

N = 10000
E = 320000
D = 128
H = 256

NC = 2
NS = 16
NW = NC * NS
CHUNK = 112
CHUNKS_PER_W = 90
E_PER_W = CHUNK * CHUNKS_PER_W
E_PAD = E_PER_W * NW
ROWS_MAIN = 624
ROWS_LAST = N - 15 * ROWS_MAIN
X_PAD_ROWS = 8
EPS = 1e-5


def _sc_partial_segment_sum(x_p, src_w, dst_w, zeros):
    mesh = plsc.VectorSubcoreMesh(core_axis_name="c", subcore_axis_name="s")

    @functools.partial(
        pl.kernel,
        out_type=jax.ShapeDtypeStruct((NC * N, D), jnp.float32),
        mesh=mesh,
        scratch_types=[
            pltpu.VMEM((E_PER_W,), jnp.int32),
            pltpu.VMEM((CHUNKS_PER_W, CHUNK), jnp.int32),
            pltpu.VMEM((2, CHUNK, D), jnp.float32),
            pltpu.VMEM_SHARED((N, D), jnp.float32),
            pltpu.SemaphoreType.DMA,
            pltpu.SemaphoreType.DMA,
        ],
    )
    def sc_kernel(x_hbm, src_hbm, dst_hbm, z_hbm, out_hbm,
                  src_v, dst_v, rows_v, acc, sem0, sem1):
        c = lax.axis_index("c")
        s = lax.axis_index("s")
        wid = s * NC + c
        r0 = s * ROWS_MAIN
        @pl.when(s < NS - 1)
        def _():
            pltpu.sync_copy(z_hbm.at[pl.ds(r0, ROWS_MAIN)],
                            acc.at[pl.ds(r0, ROWS_MAIN)])

        @pl.when(s == NS - 1)
        def _():
            pltpu.sync_copy(z_hbm.at[pl.ds(15 * ROWS_MAIN, ROWS_LAST)],
                            acc.at[pl.ds(15 * ROWS_MAIN, ROWS_LAST)])

        pltpu.sync_copy(src_hbm.at[wid], src_v)
        pltpu.sync_copy(dst_hbm.at[wid], dst_v)
        plsc.subcore_barrier()

        sems = (sem0, sem1)
        for b in range(2):
            pltpu.async_copy(x_hbm.at[src_v.at[pl.ds(b * CHUNK, CHUNK)]],
                             rows_v.at[b], sems[b])

        def body(i, carry):
            for b in range(2):
                j = 2 * i + b
                pltpu.make_async_copy(
                    x_hbm.at[src_v.at[pl.ds(j * CHUNK, CHUNK)]],
                    rows_v.at[b], sems[b]).wait()
                pltpu.sync_copy(rows_v.at[b], acc.at[dst_v.at[j]], add=True)
                nxt = j + 2

                @pl.when(nxt < CHUNKS_PER_W)
                def _():
                    pltpu.async_copy(
                        x_hbm.at[src_v.at[pl.ds(nxt * CHUNK, CHUNK)]],
                        rows_v.at[b], sems[b])
            return carry

        lax.fori_loop(0, CHUNKS_PER_W // 2, body, 0)
        plsc.subcore_barrier()

        @pl.when(s < NS - 1)
        def _():
            pltpu.sync_copy(acc.at[pl.ds(r0, ROWS_MAIN)],
                            out_hbm.at[pl.ds(c * N + r0, ROWS_MAIN)])

        @pl.when(s == NS - 1)
        def _():
            pltpu.sync_copy(acc.at[pl.ds(15 * ROWS_MAIN, ROWS_LAST)],
                            out_hbm.at[pl.ds(c * N + 15 * ROWS_MAIN, ROWS_LAST)])

    return sc_kernel(x_p, src_w, dst_w, zeros)


def _tc_dense(x, p0, p1, W_root, W_nbr, b_base, gamma1, beta1,
              W1, b1, W2, b2, gamma2, beta2):
    def body(x_ref, p0_ref, p1_ref, wr_ref, wn_ref, bb_ref, g1_ref, be1_ref,
             w1_ref, b1_ref, w2_ref, b2_ref, g2_ref, be2_ref, o_ref):
        xv = x_ref[...]
        agg = p0_ref[...] + p1_ref[...]
        h = jnp.dot(xv, wr_ref[...], preferred_element_type=jnp.float32)
        h = h + jnp.dot(agg, wn_ref[...], preferred_element_type=jnp.float32)
        h = h + bb_ref[...]
        mu = jnp.mean(h, axis=0, keepdims=True)
        hc = h - mu
        var = jnp.mean(hc * hc, axis=0, keepdims=True)
        h = hc * lax.rsqrt(var + EPS) * g1_ref[...] + be1_ref[...]
        h = jnp.maximum(h, 0.0) + xv
        t = jnp.maximum(
            jnp.dot(h, w1_ref[...], preferred_element_type=jnp.float32)
            + b1_ref[...], 0.0)
        y = (jnp.dot(t, w2_ref[...], preferred_element_type=jnp.float32)
             + b2_ref[...] + h)
        mu2 = jnp.mean(y, axis=0, keepdims=True)
        yc = y - mu2
        var2 = jnp.mean(yc * yc, axis=0, keepdims=True)
        o_ref[...] = yc * lax.rsqrt(var2 + EPS) * g2_ref[...] + be2_ref[...]

    return pl.pallas_call(
        body,
        out_shape=jax.ShapeDtypeStruct((N, D), jnp.float32),
    )(x, p0, p1, W_root, W_nbr,
      b_base.reshape(1, D), gamma1.reshape(1, D), beta1.reshape(1, D),
      W1, b1.reshape(1, H), W2, b2.reshape(1, D),
      gamma2.reshape(1, D), beta2.reshape(1, D))


def kernel(x, edge_index, W_root, W_nbr, b_base, gamma1, beta1,
           W1, b1, W2, b2, gamma2, beta2):
    src = edge_index[0]
    dst = edge_index[1]
    pad = E_PAD - E
    src_w = jnp.concatenate(
        [src, jnp.full((pad,), N, jnp.int32)]).reshape(NW, E_PER_W)
    dst_w = jnp.concatenate(
        [dst, jnp.zeros((pad,), jnp.int32)]).reshape(NW, CHUNKS_PER_W, CHUNK)
    x_p = jnp.concatenate([x, jnp.zeros((X_PAD_ROWS, D), jnp.float32)])
    zeros = jnp.zeros((N, D), jnp.float32)
    parts = _sc_partial_segment_sum(x_p, src_w, dst_w, zeros)
    p0 = parts[:N]
    p1 = parts[N:2 * N]
    return _tc_dense(x, p0, p1, W_root, W_nbr, b_base, gamma1, beta1,
                     W1, b1, W2, b2, gamma2, beta2)


# speedup vs baseline: 1.7211x; 1.0008x over previous
import functools

import jax
import jax.numpy as jnp
from jax import lax
from jax.experimental import pallas as pl
from jax.experimental.pallas import tpu as pltpu
from jax.experimental.pallas import tpu_sc as plsc

N = 10000
E = 320000
D = 128
H = 256

NC = 2
NS = 16
NW = NC * NS
CHUNK = 112
CHUNKS_PER_W = 90
E_PER_W = CHUNK * CHUNKS_PER_W
E_PAD = E_PER_W * NW
ROWS_MAIN = 624
ROWS_LAST = N - 15 * ROWS_MAIN
X_PAD_ROWS = 8
EPS = 1e-5


def _sc_partial_segment_sum(x_p, src_w, dst_w, zeros):
    mesh = plsc.VectorSubcoreMesh(core_axis_name="c", subcore_axis_name="s")

    @functools.partial(
        pl.kernel,
        out_type=jax.ShapeDtypeStruct((NC * N, D), jnp.float32),
        mesh=mesh,
        scratch_types=[
            pltpu.VMEM((E_PER_W,), jnp.int32),
            pltpu.VMEM((CHUNKS_PER_W, CHUNK), jnp.int32),
            pltpu.VMEM((2, CHUNK, D), jnp.float32),
            pltpu.VMEM_SHARED((N, D), jnp.float32),
            pltpu.SemaphoreType.DMA,
            pltpu.SemaphoreType.DMA,
        ],
    )
    def sc_kernel(x_hbm, src_hbm, dst_hbm, z_hbm, out_hbm,
                  src_v, dst_v, rows_v, acc, sem0, sem1):
        c = lax.axis_index("c")
        s = lax.axis_index("s")
        wid = s * NC + c
        r0 = s * ROWS_MAIN
        @pl.when(s < NS - 1)
        def _():
            pltpu.sync_copy(z_hbm.at[pl.ds(r0, ROWS_MAIN)],
                            acc.at[pl.ds(r0, ROWS_MAIN)])

        @pl.when(s == NS - 1)
        def _():
            pltpu.sync_copy(z_hbm.at[pl.ds(15 * ROWS_MAIN, ROWS_LAST)],
                            acc.at[pl.ds(15 * ROWS_MAIN, ROWS_LAST)])

        pltpu.sync_copy(src_hbm.at[wid], src_v)
        pltpu.sync_copy(dst_hbm.at[wid], dst_v)
        plsc.subcore_barrier()

        sems = (sem0, sem1)
        HALF = CHUNK // 2

        def start_gather(j, b):
            pltpu.async_copy(
                x_hbm.at[src_v.at[pl.ds(j * CHUNK, HALF)]],
                rows_v.at[b, pl.ds(0, HALF)], sems[b])
            pltpu.async_copy(
                x_hbm.at[src_v.at[pl.ds(j * CHUNK + HALF, HALF)]],
                rows_v.at[b, pl.ds(HALF, HALF)], sems[b])

        for b in range(2):
            start_gather(b, b)

        def body(i, carry):
            for b in range(2):
                j = 2 * i + b
                pltpu.make_async_copy(
                    x_hbm.at[src_v.at[pl.ds(j * CHUNK, CHUNK)]],
                    rows_v.at[b], sems[b]).wait()
                pltpu.sync_copy(rows_v.at[b], acc.at[dst_v.at[j]], add=True)
                nxt = j + 2

                @pl.when(nxt < CHUNKS_PER_W)
                def _():
                    start_gather(nxt, b)
            return carry

        lax.fori_loop(0, CHUNKS_PER_W // 2, body, 0)
        plsc.subcore_barrier()

        @pl.when(s < NS - 1)
        def _():
            pltpu.sync_copy(acc.at[pl.ds(r0, ROWS_MAIN)],
                            out_hbm.at[pl.ds(c * N + r0, ROWS_MAIN)])

        @pl.when(s == NS - 1)
        def _():
            pltpu.sync_copy(acc.at[pl.ds(15 * ROWS_MAIN, ROWS_LAST)],
                            out_hbm.at[pl.ds(c * N + 15 * ROWS_MAIN, ROWS_LAST)])

    return sc_kernel(x_p, src_w, dst_w, zeros)


def _tc_dense(x, p0, p1, W_root, W_nbr, b_base, gamma1, beta1,
              W1, b1, W2, b2, gamma2, beta2):
    def body(x_ref, p0_ref, p1_ref, wr_ref, wn_ref, bb_ref, g1_ref, be1_ref,
             w1_ref, b1_ref, w2_ref, b2_ref, g2_ref, be2_ref, o_ref):
        xv = x_ref[...]
        agg = p0_ref[...] + p1_ref[...]
        h = jnp.dot(xv, wr_ref[...], preferred_element_type=jnp.float32)
        h = h + jnp.dot(agg, wn_ref[...], preferred_element_type=jnp.float32)
        h = h + bb_ref[...]
        mu = jnp.mean(h, axis=0, keepdims=True)
        hc = h - mu
        var = jnp.mean(hc * hc, axis=0, keepdims=True)
        h = hc * lax.rsqrt(var + EPS) * g1_ref[...] + be1_ref[...]
        h = jnp.maximum(h, 0.0) + xv
        t = jnp.maximum(
            jnp.dot(h, w1_ref[...], preferred_element_type=jnp.float32)
            + b1_ref[...], 0.0)
        y = (jnp.dot(t, w2_ref[...], preferred_element_type=jnp.float32)
             + b2_ref[...] + h)
        mu2 = jnp.mean(y, axis=0, keepdims=True)
        yc = y - mu2
        var2 = jnp.mean(yc * yc, axis=0, keepdims=True)
        o_ref[...] = yc * lax.rsqrt(var2 + EPS) * g2_ref[...] + be2_ref[...]

    return pl.pallas_call(
        body,
        out_shape=jax.ShapeDtypeStruct((N, D), jnp.float32),
    )(x, p0, p1, W_root, W_nbr,
      b_base.reshape(1, D), gamma1.reshape(1, D), beta1.reshape(1, D),
      W1, b1.reshape(1, H), W2, b2.reshape(1, D),
      gamma2.reshape(1, D), beta2.reshape(1, D))


def kernel(x, edge_index, W_root, W_nbr, b_base, gamma1, beta1,
           W1, b1, W2, b2, gamma2, beta2):
    src = edge_index[0]
    dst = edge_index[1]
    pad = E_PAD - E
    src_w = jnp.concatenate(
        [src, jnp.full((pad,), N, jnp.int32)]).reshape(NW, E_PER_W)
    dst_w = jnp.concatenate(
        [dst, jnp.zeros((pad,), jnp.int32)]).reshape(NW, CHUNKS_PER_W, CHUNK)
    x_p = jnp.concatenate([x, jnp.zeros((X_PAD_ROWS, D), jnp.float32)])
    zeros = jnp.zeros((N, D), jnp.float32)
    parts = _sc_partial_segment_sum(x_p, src_w, dst_w, zeros)
    p0 = parts[:N]
    p1 = parts[N:2 * N]
    return _tc_dense(x, p0, p1, W_root, W_nbr, b_base, gamma1, beta1,
                     W1, b1, W2, b2, gamma2, beta2)


# speedup vs baseline: 1.7320x; 1.0063x over previous
import functools

import jax
import jax.numpy as jnp
from jax import lax
from jax.experimental import pallas as pl
from jax.experimental.pallas import tpu as pltpu
from jax.experimental.pallas import tpu_sc as plsc

N = 10000
E = 320000
D = 128
H = 256

NC = 2
NS = 16
NW = NC * NS
CHUNK = 72
PAIR_CHUNKS = 280
K0 = 120
K1 = PAIR_CHUNKS - K0
TOTAL_CHUNKS = NS * PAIR_CHUNKS
E_PER_PAIR = CHUNK * PAIR_CHUNKS
E_PAD = E_PER_PAIR * NS
ROWS_MAIN = 624
ROWS_LAST = N - 15 * ROWS_MAIN
X_PAD_ROWS = 8
EPS = 1e-5


def _sc_partial_segment_sum(x_p, src_w, dst_w, zeros):
    mesh = plsc.VectorSubcoreMesh(core_axis_name="c", subcore_axis_name="s")

    @functools.partial(
        pl.kernel,
        out_type=jax.ShapeDtypeStruct((NC * N, D), jnp.float32),
        mesh=mesh,
        scratch_types=[
            pltpu.VMEM((K1 * CHUNK,), jnp.int32),
            pltpu.VMEM((K1, CHUNK), jnp.int32),
            pltpu.VMEM((2, CHUNK, D), jnp.float32),
            pltpu.VMEM_SHARED((N, D), jnp.float32),
            pltpu.SemaphoreType.DMA,
            pltpu.SemaphoreType.DMA,
        ],
    )
    def sc_kernel(x_hbm, src_hbm, dst_hbm, z_hbm, out_hbm,
                  src_v, dst_v, rows_v, acc, sem0, sem1):
        c = lax.axis_index("c")
        s = lax.axis_index("s")
        r0 = s * ROWS_MAIN
        base_chunk = s * PAIR_CHUNKS + c * K0
        nchunks = jnp.where(c == 0, K0, K1)
        @pl.when(s < NS - 1)
        def _():
            pltpu.sync_copy(z_hbm.at[pl.ds(r0, ROWS_MAIN)],
                            acc.at[pl.ds(r0, ROWS_MAIN)])

        @pl.when(s == NS - 1)
        def _():
            pltpu.sync_copy(z_hbm.at[pl.ds(15 * ROWS_MAIN, ROWS_LAST)],
                            acc.at[pl.ds(15 * ROWS_MAIN, ROWS_LAST)])

        @pl.when(c == 0)
        def _():
            pltpu.sync_copy(src_hbm.at[pl.ds(base_chunk * CHUNK, K0 * CHUNK)],
                            src_v.at[pl.ds(0, K0 * CHUNK)])
            pltpu.sync_copy(dst_hbm.at[pl.ds(base_chunk, K0)],
                            dst_v.at[pl.ds(0, K0)])

        @pl.when(c == 1)
        def _():
            pltpu.sync_copy(src_hbm.at[pl.ds(base_chunk * CHUNK, K1 * CHUNK)],
                            src_v)
            pltpu.sync_copy(dst_hbm.at[pl.ds(base_chunk, K1)], dst_v)
        plsc.subcore_barrier()

        sems = (sem0, sem1)

        def start_gather(j, b):
            pltpu.async_copy(x_hbm.at[src_v.at[pl.ds(j * CHUNK, CHUNK)]],
                             rows_v.at[b], sems[b])

        for b in range(2):
            start_gather(b, b)

        def body(i, carry):
            for b in range(2):
                j = 2 * i + b
                pltpu.make_async_copy(
                    x_hbm.at[src_v.at[pl.ds(j * CHUNK, CHUNK)]],
                    rows_v.at[b], sems[b]).wait()
                pltpu.sync_copy(rows_v.at[b], acc.at[dst_v.at[j]], add=True)
                nxt = j + 2

                @pl.when(nxt < nchunks)
                def _():
                    start_gather(nxt, b)
            return carry

        lax.fori_loop(0, nchunks // 2, body, 0)
        plsc.subcore_barrier()

        @pl.when(s < NS - 1)
        def _():
            pltpu.sync_copy(acc.at[pl.ds(r0, ROWS_MAIN)],
                            out_hbm.at[pl.ds(c * N + r0, ROWS_MAIN)])

        @pl.when(s == NS - 1)
        def _():
            pltpu.sync_copy(acc.at[pl.ds(15 * ROWS_MAIN, ROWS_LAST)],
                            out_hbm.at[pl.ds(c * N + 15 * ROWS_MAIN, ROWS_LAST)])

    return sc_kernel(x_p, src_w, dst_w, zeros)


def _tc_dense(x, p0, p1, W_root, W_nbr, b_base, gamma1, beta1,
              W1, b1, W2, b2, gamma2, beta2):
    def body(x_ref, p0_ref, p1_ref, wr_ref, wn_ref, bb_ref, g1_ref, be1_ref,
             w1_ref, b1_ref, w2_ref, b2_ref, g2_ref, be2_ref, o_ref):
        xv = x_ref[...]
        agg = p0_ref[...] + p1_ref[...]
        h = jnp.dot(xv, wr_ref[...], preferred_element_type=jnp.float32)
        h = h + jnp.dot(agg, wn_ref[...], preferred_element_type=jnp.float32)
        h = h + bb_ref[...]
        mu = jnp.mean(h, axis=0, keepdims=True)
        hc = h - mu
        var = jnp.mean(hc * hc, axis=0, keepdims=True)
        h = hc * lax.rsqrt(var + EPS) * g1_ref[...] + be1_ref[...]
        h = jnp.maximum(h, 0.0) + xv
        t = jnp.maximum(
            jnp.dot(h, w1_ref[...], preferred_element_type=jnp.float32)
            + b1_ref[...], 0.0)
        y = (jnp.dot(t, w2_ref[...], preferred_element_type=jnp.float32)
             + b2_ref[...] + h)
        mu2 = jnp.mean(y, axis=0, keepdims=True)
        yc = y - mu2
        var2 = jnp.mean(yc * yc, axis=0, keepdims=True)
        o_ref[...] = yc * lax.rsqrt(var2 + EPS) * g2_ref[...] + be2_ref[...]

    return pl.pallas_call(
        body,
        out_shape=jax.ShapeDtypeStruct((N, D), jnp.float32),
    )(x, p0, p1, W_root, W_nbr,
      b_base.reshape(1, D), gamma1.reshape(1, D), beta1.reshape(1, D),
      W1, b1.reshape(1, H), W2, b2.reshape(1, D),
      gamma2.reshape(1, D), beta2.reshape(1, D))


def kernel(x, edge_index, W_root, W_nbr, b_base, gamma1, beta1,
           W1, b1, W2, b2, gamma2, beta2):
    src = edge_index[0]
    dst = edge_index[1]
    pad = E_PAD - E
    src_w = jnp.concatenate([src, jnp.full((pad,), N, jnp.int32)])
    dst_w = jnp.concatenate(
        [dst, jnp.zeros((pad,), jnp.int32)]).reshape(TOTAL_CHUNKS, CHUNK)
    x_p = jnp.concatenate([x, jnp.zeros((X_PAD_ROWS, D), jnp.float32)])
    zeros = jnp.zeros((N, D), jnp.float32)
    parts = _sc_partial_segment_sum(x_p, src_w, dst_w, zeros)
    p0 = parts[:N]
    p1 = parts[N:2 * N]
    return _tc_dense(x, p0, p1, W_root, W_nbr, b_base, gamma1, beta1,
                     W1, b1, W2, b2, gamma2, beta2)
